# baseline (speedup 1.0000x reference)
import functools

import jax
import jax.numpy as jnp
from jax import lax
from jax.experimental import pallas as pl
from jax.experimental.pallas import tpu as pltpu
from jax.experimental.pallas import tpu_sc as plsc

NC = 2
NS = 16
NW = NC * NS
CHUNK = 128


def _build(N, E, D, H):
  NPG = -(-(N + 1) // (NS * 8)) * NS * 8
  ZR = N
  N_ACC = ((NPG + NW * 8 - 1) // (NW * 8)) * NW * 8
  CPW = -(-E // (NW * CHUNK))
  EPAD = NW * CHUNK * CPW
  ROWS_PER_TILE = NPG // NS
  ACC_PER_TILE = N_ACC // NS

  mesh = plsc.VectorSubcoreMesh(core_axis_name="c", subcore_axis_name="s",
                                num_cores=NC, num_subcores=NS)

  def make_sc_agg(W):
    @functools.partial(
        pl.kernel,
        out_type=jax.ShapeDtypeStruct((NC, NPG, W), jnp.float32),
        mesh=mesh,
        scratch_types=[
            pltpu.VMEM((CHUNK,), jnp.int32),
            pltpu.VMEM((CHUNK,), jnp.int32),
            pltpu.VMEM((CHUNK, W), jnp.float32),
            pltpu.VMEM((CHUNK, W), jnp.float32),
            pltpu.VMEM_SHARED((N_ACC, W), jnp.float32),
            pltpu.SemaphoreType.DMA,
        ],
    )
    def agg_kernel(g_hbm, srcp_hbm, dst_hbm, out_hbm, sidx, didx, rows, zbuf,
                   acc, sem):
      c = lax.axis_index("c")
      s = lax.axis_index("s")
      wid = c * NS + s

      def zrow(i, _):
        for j in range(W // 16):
          zbuf[i, pl.ds(16 * j, 16)] = jnp.zeros((16,), jnp.float32)
        return 0
      lax.fori_loop(0, CHUNK, zrow, 0)
      for t in range(ACC_PER_TILE // CHUNK):
        pltpu.sync_copy(zbuf, acc.at[pl.ds(s * ACC_PER_TILE + t * CHUNK,
                                           CHUNK)])
      rem = ACC_PER_TILE % CHUNK
      if rem:
        pltpu.sync_copy(
            zbuf.at[pl.ds(0, rem)],
            acc.at[pl.ds(s * ACC_PER_TILE + (ACC_PER_TILE // CHUNK) * CHUNK,
                         rem)])
      plsc.subcore_barrier()

      def body(t, _):
        off = pl.multiple_of((wid * CPW + t) * CHUNK, CHUNK)
        pltpu.sync_copy(dst_hbm.at[pl.ds(off, CHUNK)], didx)
        pltpu.sync_copy(srcp_hbm.at[pl.ds(off, CHUNK)], sidx)
        pltpu.async_copy(g_hbm.at[sidx], rows, sem).wait()
        pltpu.sync_copy(rows, acc.at[didx], add=True)
        return 0
      lax.fori_loop(0, CPW, body, 0)
      plsc.subcore_barrier()

      for t in range(-(-ROWS_PER_TILE // CHUNK)):
        r0 = s * ROWS_PER_TILE + t * CHUNK
        cnt = min(CHUNK, ROWS_PER_TILE - t * CHUNK)
        pltpu.sync_copy(acc.at[pl.ds(r0, cnt)],
                        out_hbm.at[c, pl.ds(r0, cnt)])

    return agg_kernel

  agg_kernel = make_sc_agg(H)

  def dinv_of(dega):
    d = 1.0 + dega[0, :, 0:1] + dega[1, :, 0:1]
    row = lax.broadcasted_iota(jnp.int32, (NPG, 1), 0)
    return jnp.where(row < N, lax.rsqrt(d), 0.0)

  def tc_first_body(xp_ref, w_ref, dega_ref, g_ref):
    dinv = dinv_of(dega_ref[...])
    h = jnp.dot(xp_ref[...], w_ref[...], preferred_element_type=jnp.float32)
    g_ref[...] = dinv * h

  def tc_mid_body(a_ref, g_ref, dega_ref, b_ref, w_ref, gn_ref):
    dinv = dinv_of(dega_ref[...])
    a = a_ref[0] + a_ref[1] + g_ref[...]
    x = jax.nn.relu(dinv * a + b_ref[...])
    gn_ref[...] = dinv * jnp.dot(x, w_ref[...],
                                 preferred_element_type=jnp.float32)

  def tc_last_body(a_ref, g_ref, dega_ref, b_ref, cw1_ref, cb1_ref, cw2_ref,
                   cb2_ref, out_ref):
    dinv = dinv_of(dega_ref[...])
    a = a_ref[0] + a_ref[1] + g_ref[...]
    x = jax.nn.relu(dinv * a + b_ref[...])
    hc = jax.nn.relu(
        jnp.dot(x, cw1_ref[...], preferred_element_type=jnp.float32)
        + cb1_ref[...])
    o = jnp.dot(hc, cw2_ref[...], preferred_element_type=jnp.float32)
    out_ref[...] = jnp.broadcast_to(o + cb2_ref[...], (NPG, H))

  fsd = jax.ShapeDtypeStruct
  tc_first = pl.pallas_call(tc_first_body, out_shape=fsd((NPG, H),
                                                         jnp.float32))
  tc_mid = pl.pallas_call(tc_mid_body, out_shape=fsd((NPG, H), jnp.float32))
  tc_last = pl.pallas_call(tc_last_body, out_shape=fsd((NPG, H),
                                                       jnp.float32))

  def run(x, edge_index, W1, b1, W2, b2, W3, b3, CW1, Cb1, CW2, Cb2):
    ei = edge_index.astype(jnp.int32)
    key = ei[1] * N + ei[0]
    sk = jnp.sort(key)
    first = jnp.concatenate(
        [jnp.ones((1,), bool), sk[1:] != sk[:-1]])
    src = sk % N
    dst = sk // N
    srcp = jnp.where(first, src, ZR)
    pad_e = EPAD - E
    srcp = jnp.concatenate([srcp, jnp.full((pad_e,), ZR, jnp.int32)])
    dstp = jnp.concatenate([dst, jnp.full((pad_e,), ZR, jnp.int32)])
    widx = jnp.concatenate(
        [jnp.where(first, 0, 1).astype(jnp.int32),
         jnp.ones((pad_e,), jnp.int32)])
    onehot = jnp.zeros((8, H), jnp.float32).at[0].set(1.0)

    xp = jnp.concatenate([x, jnp.zeros((NPG - N, D), jnp.float32)])

    dega = agg_kernel(onehot, widx, dstp)
    g1 = tc_first(xp, W1, dega)
    a1 = agg_kernel(g1, srcp, dstp)
    g2 = tc_mid(a1, g1, dega, b1.reshape(1, H), W2)
    a2 = agg_kernel(g2, srcp, dstp)
    g3 = tc_mid(a2, g2, dega, b2.reshape(1, H), W3)
    a3 = agg_kernel(g3, srcp, dstp)
    out = tc_last(a3, g3, dega, b3.reshape(1, H), CW1, Cb1.reshape(1, H),
                  CW2, Cb2.reshape(1, 1))
    return out[:N, 0]

  return run


def kernel(x, edge_index, W1, b1, W2, b2, W3, b3, CW1, Cb1, CW2, Cb2):
  N, D = x.shape
  E = edge_index.shape[1]
  H = W1.shape[1]
  run = _build(N, E, D, H)
  return run(x, edge_index, W1, b1, W2, b2, W3, b3, CW1, Cb1, CW2, Cb2)


# speedup vs baseline: 7.3780x; 7.3780x over previous
import functools

import jax
import jax.numpy as jnp
from jax import lax
from jax.experimental import pallas as pl
from jax.experimental.pallas import tpu as pltpu
from jax.experimental.pallas import tpu_sc as plsc

NC = 2
NS = 16
NW = NC * NS
CHUNK = 128


def _build(N, E, D, H):
  NPG = -(-(N + 1) // (NS * 8)) * NS * 8
  ZR = N
  N_ACC = ((NPG + NW * 8 - 1) // (NW * 8)) * NW * 8
  CPW = -(-E // (NW * CHUNK))
  EPAD = NW * CHUNK * CPW
  ROWS_PER_TILE = NPG // NS
  ACC_PER_TILE = N_ACC // NS

  mesh = plsc.VectorSubcoreMesh(core_axis_name="c", subcore_axis_name="s",
                                num_cores=NC, num_subcores=NS)

  def make_sc_agg(W):
    @functools.partial(
        pl.kernel,
        out_type=jax.ShapeDtypeStruct((NC, NPG, W), jnp.float32),
        mesh=mesh,
        scratch_types=[
            pltpu.VMEM((CHUNK,), jnp.int32),
            pltpu.VMEM((CHUNK,), jnp.int32),
            pltpu.VMEM((CHUNK, W), jnp.float32),
            pltpu.VMEM((CHUNK, W), jnp.float32),
            pltpu.VMEM_SHARED((N_ACC, W), jnp.float32),
            pltpu.SemaphoreType.DMA,
        ],
    )
    def agg_kernel(g_hbm, srcp_hbm, dst_hbm, out_hbm, sidx, didx, rows, zbuf,
                   acc, sem):
      c = lax.axis_index("c")
      s = lax.axis_index("s")
      wid = c * NS + s

      def zrow(i, _):
        for j in range(W // 16):
          zbuf[i, pl.ds(16 * j, 16)] = jnp.zeros((16,), jnp.float32)
        return 0
      lax.fori_loop(0, CHUNK, zrow, 0)
      for t in range(ACC_PER_TILE // CHUNK):
        pltpu.sync_copy(zbuf, acc.at[pl.ds(s * ACC_PER_TILE + t * CHUNK,
                                           CHUNK)])
      rem = ACC_PER_TILE % CHUNK
      if rem:
        pltpu.sync_copy(
            zbuf.at[pl.ds(0, rem)],
            acc.at[pl.ds(s * ACC_PER_TILE + (ACC_PER_TILE // CHUNK) * CHUNK,
                         rem)])
      plsc.subcore_barrier()

      def body(t, _):
        off = pl.multiple_of((wid * CPW + t) * CHUNK, CHUNK)
        pltpu.sync_copy(dst_hbm.at[pl.ds(off, CHUNK)], didx)
        pltpu.sync_copy(srcp_hbm.at[pl.ds(off, CHUNK)], sidx)
        pltpu.async_copy(g_hbm.at[sidx], rows, sem).wait()
        pltpu.sync_copy(rows, acc.at[didx], add=True)
        return 0
      lax.fori_loop(0, CPW, body, 0)
      plsc.subcore_barrier()

      for t in range(-(-ROWS_PER_TILE // CHUNK)):
        r0 = s * ROWS_PER_TILE + t * CHUNK
        cnt = min(CHUNK, ROWS_PER_TILE - t * CHUNK)
        pltpu.sync_copy(acc.at[pl.ds(r0, cnt)],
                        out_hbm.at[c, pl.ds(r0, cnt)])

    return agg_kernel

  agg_kernel = make_sc_agg(H)

  def dinv_of(dega):
    d = 1.0 + dega[0, :, 0:1] + dega[1, :, 0:1]
    row = lax.broadcasted_iota(jnp.int32, (NPG, 1), 0)
    return jnp.where(row < N, lax.rsqrt(d), 0.0)

  def tc_first_body(xp_ref, w_ref, dega_ref, g_ref):
    dinv = dinv_of(dega_ref[...])
    h = jnp.dot(xp_ref[...], w_ref[...], preferred_element_type=jnp.float32)
    g_ref[...] = dinv * h

  def tc_mid_body(a_ref, g_ref, dega_ref, b_ref, w_ref, gn_ref):
    dinv = dinv_of(dega_ref[...])
    a = a_ref[0] + a_ref[1] + g_ref[...]
    x = jax.nn.relu(dinv * a + b_ref[...])
    gn_ref[...] = dinv * jnp.dot(x, w_ref[...],
                                 preferred_element_type=jnp.float32)

  def tc_last_body(a_ref, g_ref, dega_ref, b_ref, cw1_ref, cb1_ref, cw2_ref,
                   cb2_ref, out_ref):
    dinv = dinv_of(dega_ref[...])
    a = a_ref[0] + a_ref[1] + g_ref[...]
    x = jax.nn.relu(dinv * a + b_ref[...])
    hc = jax.nn.relu(
        jnp.dot(x, cw1_ref[...], preferred_element_type=jnp.float32)
        + cb1_ref[...])
    o = jnp.dot(hc, cw2_ref[...], preferred_element_type=jnp.float32)
    out_ref[...] = jnp.broadcast_to(o + cb2_ref[...], (NPG, H))

  fsd = jax.ShapeDtypeStruct
  tc_first = pl.pallas_call(tc_first_body, out_shape=fsd((NPG, H),
                                                         jnp.float32))
  tc_mid = pl.pallas_call(tc_mid_body, out_shape=fsd((NPG, H), jnp.float32))
  tc_last = pl.pallas_call(tc_last_body, out_shape=fsd((NPG, H),
                                                       jnp.float32))

  def run(x, edge_index, W1, b1, W2, b2, W3, b3, CW1, Cb1, CW2, Cb2):
    ei = edge_index.astype(jnp.int32)
    key = ei[1] * N + ei[0]
    sk = jnp.sort(key)
    first = jnp.concatenate(
        [jnp.ones((1,), bool), sk[1:] != sk[:-1]])
    src = sk % N
    dst = sk // N
    srcp = jnp.where(first, src, ZR)
    pad_e = EPAD - E
    srcp = jnp.concatenate([srcp, jnp.full((pad_e,), ZR, jnp.int32)])
    dstp = jnp.concatenate([dst, jnp.full((pad_e,), ZR, jnp.int32)])
    spread = jnp.arange(EPAD, dtype=jnp.int32) % 128
    widx = spread + jnp.where(
        jnp.concatenate([first, jnp.zeros((pad_e,), bool)]), 0, 128)
    onehot = jnp.concatenate([jnp.ones((128, H), jnp.float32),
                              jnp.zeros((128, H), jnp.float32)])

    xp = jnp.concatenate([x, jnp.zeros((NPG - N, D), jnp.float32)])

    dega = agg_kernel(onehot, widx, dstp)
    g1 = tc_first(xp, W1, dega)
    a1 = agg_kernel(g1, srcp, dstp)
    g2 = tc_mid(a1, g1, dega, b1.reshape(1, H), W2)
    a2 = agg_kernel(g2, srcp, dstp)
    g3 = tc_mid(a2, g2, dega, b2.reshape(1, H), W3)
    a3 = agg_kernel(g3, srcp, dstp)
    out = tc_last(a3, g3, dega, b3.reshape(1, H), CW1, Cb1.reshape(1, H),
                  CW2, Cb2.reshape(1, 1))
    return out[:N, 0]

  return run


def kernel(x, edge_index, W1, b1, W2, b2, W3, b3, CW1, Cb1, CW2, Cb2):
  N, D = x.shape
  E = edge_index.shape[1]
  H = W1.shape[1]
  run = _build(N, E, D, H)
  return run(x, edge_index, W1, b1, W2, b2, W3, b3, CW1, Cb1, CW2, Cb2)
